# baseline (speedup 1.0000x reference)
import functools

import jax
import jax.numpy as jnp
from jax import lax
from jax.experimental import pallas as pl
from jax.experimental.pallas import tpu as pltpu
from jax.experimental.pallas import tpu_sc as plsc

_N = 1024
_F = 784
_NBR = 5
_BIG = 1e10
_STEPS = 10
_BLK = 128
_MASKED = 3.0e38

_ROWS_PER_W = _N // 32
_SLOTS = 16


def _sc_topk_kernel(dist_ref, val_ref, idx_ref, rows_v, oval_v, oidx_v):
    wid = lax.axis_index("s") * 2 + lax.axis_index("c")
    base = wid * _ROWS_PER_W
    pltpu.sync_copy(dist_ref.at[pl.ds(base * _N, _ROWS_PER_W * _N)], rows_v)
    iota = lax.iota(jnp.int32, 16)

    def row_body(r, carry):
        rowbase = r * _N
        mprev = jnp.full((16,), jnp.float32(-1.0), jnp.float32)
        gprev = jnp.full((16,), jnp.int32(-1), jnp.int32)
        vals16 = jnp.zeros((16,), jnp.float32)
        idxs16 = jnp.zeros((16,), jnp.int32)
        for t in range(_NBR + 1):
            def scan_chunk(c, acc, mprev=mprev, gprev=gprev):
                vmin, argc = acc
                off = pl.multiple_of(rowbase + c * 16, 16)
                v = rows_v[pl.ds(off, 16)]
                i16 = c * 16 + iota
                valid = jnp.logical_or(
                    v > mprev, jnp.logical_and(v == mprev, i16 > gprev))
                v = jnp.where(valid, v, jnp.float32(_MASKED))
                upd = v < vmin
                return (jnp.where(upd, v, vmin), jnp.where(upd, c, argc))

            vmin, argc = lax.fori_loop(
                0, _N // 16, scan_chunk,
                (jnp.full((16,), jnp.float32(_MASKED), jnp.float32),
                 jnp.zeros((16,), jnp.int32)))
            m16 = vmin
            for k in (8, 4, 2, 1):
                perm = jnp.bitwise_xor(iota, jnp.int32(k))
                m16 = jnp.minimum(m16, m16.at[perm].get(mode="promise_in_bounds"))
            cand = jnp.where(vmin == m16, argc * 16 + iota, jnp.int32(1 << 30))
            g16 = cand
            for k in (8, 4, 2, 1):
                perm = jnp.bitwise_xor(iota, jnp.int32(k))
                g16 = jnp.minimum(g16, g16.at[perm].get(mode="promise_in_bounds"))
            mprev, gprev = m16, g16
            vals16 = jnp.where(iota == t, m16, vals16)
            idxs16 = jnp.where(iota == t, g16, idxs16)
        obase = pl.multiple_of(r * _SLOTS, 16)
        oval_v[pl.ds(obase, 16)] = vals16
        oidx_v[pl.ds(obase, 16)] = idxs16
        return carry

    lax.fori_loop(0, _ROWS_PER_W, row_body, jnp.int32(0))
    pltpu.sync_copy(oval_v, val_ref.at[pl.ds(base * _SLOTS, _ROWS_PER_W * _SLOTS)])
    pltpu.sync_copy(oidx_v, idx_ref.at[pl.ds(base * _SLOTS, _ROWS_PER_W * _SLOTS)])


def _sc_topk(dist):
    fn = pl.kernel(
        _sc_topk_kernel,
        out_type=[
            jax.ShapeDtypeStruct((_N * _SLOTS,), jnp.float32),
            jax.ShapeDtypeStruct((_N * _SLOTS,), jnp.int32),
        ],
        mesh=plsc.VectorSubcoreMesh(core_axis_name="c", subcore_axis_name="s"),
        scratch_types=[
            pltpu.VMEM((_ROWS_PER_W * _N,), jnp.float32),
            pltpu.VMEM((_ROWS_PER_W * _SLOTS,), jnp.float32),
            pltpu.VMEM((_ROWS_PER_W * _SLOTS,), jnp.int32),
        ],
    )
    return fn(dist.reshape(_N * _N))


def _graph_kernel(idx_ref, valr_ref, g_ref):
    col_ids = jax.lax.broadcasted_iota(jnp.int32, (_N, _N), 1)
    g = jnp.full((_N, _N), _BIG, jnp.float32)
    for t in range(1, _NBR + 1):
        oh = col_ids == idx_ref[:, t:t + 1]
        g = jnp.where(oh, valr_ref[:, t:t + 1], g)
    g = jnp.minimum(g, g.T)
    row_ids = jax.lax.broadcasted_iota(jnp.int32, (_N, _N), 0)
    g = jnp.where(row_ids == col_ids, jnp.float32(0.0), g)
    g_ref[...] = g


def _minplus_kernel(ga_ref, gf_ref, h_ref, ch_ref):
    i = pl.program_id(0)
    ga = ga_ref[...]
    ch = 128
    h = jnp.full((_BLK, _N), jnp.float32(jnp.inf), jnp.float32)

    def body(c, h):
        base = c * ch
        a = ga_ref[:, pl.ds(base, ch)]
        b = gf_ref[pl.ds(base, ch), :]
        cands = [a[:, t:t + 1] + b[t:t + 1, :] for t in range(ch)]
        while len(cands) > 1:
            cands = [jnp.minimum(cands[2 * u], cands[2 * u + 1])
                     for u in range(len(cands) // 2)]
        return jnp.minimum(h, cands[0])

    h = jax.lax.fori_loop(0, _N // ch, body, h)
    h_ref[...] = h
    changed = jnp.max(jnp.where(h < ga, jnp.float32(1.0), jnp.float32(0.0)))

    @pl.when(i == 0)
    def _init():
        ch_ref[0, 0] = jnp.float32(0.0)

    ch_ref[0, 0] = jnp.maximum(ch_ref[0, 0], changed)


def _minplus_call(g):
    h, chg = pl.pallas_call(
        _minplus_kernel,
        grid=(_N // _BLK,),
        in_specs=[
            pl.BlockSpec((_BLK, _N), lambda i: (i, 0)),
            pl.BlockSpec((_N, _N), lambda i: (0, 0)),
        ],
        out_specs=[
            pl.BlockSpec((_BLK, _N), lambda i: (i, 0)),
            pl.BlockSpec((1, 1), lambda i: (0, 0), memory_space=pltpu.SMEM),
        ],
        out_shape=[
            jax.ShapeDtypeStruct((_N, _N), jnp.float32),
            jax.ShapeDtypeStruct((1, 1), jnp.float32),
        ],
    )(g, g)
    return h, chg[0, 0] > 0.5


def _minplus_square_tail(D, chunk=64):
    n = D.shape[0]
    outs = []
    for s in range(0, n, chunk):
        block = D[s:s + chunk]
        cand = block[:, :, None] + D[None, :, :]
        outs.append(jnp.min(cand, axis=1))
    return jnp.concatenate(outs, axis=0)


def kernel(toLearn):
    flat = toLearn.reshape(toLearn.shape[0], -1)
    sq = jnp.sum(flat * flat, axis=1)
    d2 = sq[:, None] + sq[None, :] - 2.0 * (flat @ flat.T)
    dist = jnp.sqrt(jnp.maximum(d2, 0.0))

    vals_flat, idxs_flat = _sc_topk(dist)
    nbr_val = vals_flat.reshape(_N, _SLOTS)
    nbr_idx = idxs_flat.reshape(_N, _SLOTS)

    g = pl.pallas_call(
        _graph_kernel,
        out_shape=jax.ShapeDtypeStruct((_N, _N), jnp.float32),
    )(nbr_idx, nbr_val)

    def cond(carry):
        _, it, chg = carry
        return jnp.logical_and(it < _STEPS - 1, chg)

    def body(carry):
        gc, it, _ = carry
        h, chg = _minplus_call(gc)
        return h, it + 1, chg

    g, _, _ = jax.lax.while_loop(cond, body, (g, jnp.int32(0), jnp.bool_(True)))

    g = _minplus_square_tail(g)
    finite = g < _BIG * 0.5
    maxfin = jnp.max(jnp.where(finite, g, 0.0))
    D_geo = jnp.where(finite, g, maxfin)
    D2 = D_geo * D_geo
    row_mean = jnp.mean(D2, axis=1, keepdims=True)
    col_mean = jnp.mean(D2, axis=0, keepdims=True)
    tot = jnp.mean(D2)
    K = -0.5 * (D2 - row_mean - col_mean + tot)
    K = 0.5 * (K + K.T)
    evals, evecs = jnp.linalg.eigh(K)
    evals = evals[::-1][:_F]
    evecs = evecs[:, ::-1][:, :_F]
    emb = evecs * jnp.sqrt(jnp.maximum(evals, 0.0))[None, :]
    return emb.astype(jnp.float32)
